# baseline (speedup 1.0000x reference)
import functools

import jax
import jax.numpy as jnp
from jax import lax
from jax.experimental import pallas as pl
from jax.experimental.pallas import tpu as pltpu
from jax.experimental.pallas import tpu_sc as plsc

B = 16384
E = 16
NC = 2
NS = 16
NW = NC * NS
BPW = B // NW
CHW = 128
CH = BPW // CHW


def _sc_body(mid_hbm, tid_hbm, emg_hbm, etg_hbm, emm_hbm, etm_hbm,
             gmf_out, mm_out, tt_out,
             idx_m, idx_t, emg_v, etg_v, emm_v, etm_v, sem):
    wid = lax.axis_index("s") * NC + lax.axis_index("c")
    row0 = wid * CH
    pltpu.sync_copy(mid_hbm.at[pl.ds(row0, CH)], idx_m)
    pltpu.sync_copy(tid_hbm.at[pl.ds(row0, CH)], idx_t)
    copies = []
    for c in range(CH):
        sl = pl.ds(c * CHW, CHW)
        copies.append(pltpu.async_copy(emg_hbm.at[idx_m.at[c]], emg_v.at[sl], sem))
        copies.append(pltpu.async_copy(etg_hbm.at[idx_t.at[c]], etg_v.at[sl], sem))
        copies.append(pltpu.async_copy(emm_hbm.at[idx_m.at[c]], emm_v.at[sl], sem))
        copies.append(pltpu.async_copy(etm_hbm.at[idx_t.at[c]], etm_v.at[sl], sem))
    for cp in copies:
        cp.wait()

    def mul_body(i, carry):
        emg_v[i] = emg_v[i] * etg_v[i]
        return carry

    lax.fori_loop(0, BPW, mul_body, 0)

    base = wid * BPW
    pltpu.sync_copy(emg_v, gmf_out.at[pl.ds(base, BPW)])
    pltpu.sync_copy(emm_v, mm_out.at[pl.ds(base, BPW)])
    pltpu.sync_copy(etm_v, tt_out.at[pl.ds(base, BPW)])


@functools.cache
def _sc_gather():
    return pl.kernel(
        _sc_body,
        out_type=(
            jax.ShapeDtypeStruct((B, E), jnp.float32),
            jax.ShapeDtypeStruct((B, 2 * E), jnp.float32),
            jax.ShapeDtypeStruct((B, 2 * E), jnp.float32),
        ),
        mesh=plsc.VectorSubcoreMesh(core_axis_name="c", subcore_axis_name="s"),
        scratch_types=[
            pltpu.VMEM((CH, CHW), jnp.int32),
            pltpu.VMEM((CH, CHW), jnp.int32),
            pltpu.VMEM((BPW, E), jnp.float32),
            pltpu.VMEM((BPW, E), jnp.float32),
            pltpu.VMEM((BPW, 2 * E), jnp.float32),
            pltpu.VMEM((BPW, 2 * E), jnp.float32),
            pltpu.SemaphoreType.DMA,
        ],
        compiler_params=pltpu.CompilerParams(use_tc_tiling_on_sc=False),
    )


BLK = 2048


def _tc_body(gmf_ref, mm_ref, tt_ref, w1a, w1b, b1r, w2, b2r, w3, b3r,
             wog, wom, bor, out_ref):
    h = mm_ref[:] @ w1a[:] + tt_ref[:] @ w1b[:] + b1r[:]
    h = jnp.maximum(h, 0.0)
    h = jnp.maximum(h @ w2[:] + b2r[:], 0.0)
    h = jnp.maximum(h @ w3[:] + b3r[:], 0.0)
    out = (jnp.sum(gmf_ref[:] * wog[:], axis=-1)
           + jnp.sum(h * wom[:], axis=-1) + bor[0, 0])
    out_ref[:] = out


def _tc_mlp(gmf, mm, tt, w1a, w1b, b1r, w2, b2r, w3, b3r, wog, wom, bor):
    full = lambda shape: pl.BlockSpec(shape, lambda i: (0, 0))
    return pl.pallas_call(
        _tc_body,
        grid=(B // BLK,),
        in_specs=[
            pl.BlockSpec((BLK, E), lambda i: (i, 0)),
            pl.BlockSpec((BLK, 2 * E), lambda i: (i, 0)),
            pl.BlockSpec((BLK, 2 * E), lambda i: (i, 0)),
            full((2 * E, 64)),
            full((2 * E, 64)),
            full((1, 64)),
            full((64, 32)),
            full((1, 32)),
            full((32, 16)),
            full((1, 16)),
            full((1, E)),
            full((1, 16)),
            full((1, 1)),
        ],
        out_specs=pl.BlockSpec((BLK,), lambda i: (i,)),
        out_shape=jax.ShapeDtypeStruct((B,), jnp.float32),
    )(gmf, mm, tt, w1a, w1b, b1r, w2, b2r, w3, b3r, wog, wom, bor)


def kernel(model_ids, task_ids, Emg, Etg, Emm, Etm, W1, b1, W2, b2, W3, b3, Wo, bo):
    mid2 = model_ids.reshape(B // CHW, CHW)
    tid2 = task_ids.reshape(B // CHW, CHW)
    gmf, mm, tt = _sc_gather()(mid2, tid2, Emg, Etg, Emm, Etm)
    return _tc_mlp(
        gmf, mm, tt,
        W1[: 2 * E], W1[2 * E:], b1.reshape(1, 64),
        W2, b2.reshape(1, 32),
        W3, b3.reshape(1, 16),
        Wo[:E].reshape(1, E), Wo[E:].reshape(1, 16),
        bo.reshape(1, 1),
    )
